# baseline (speedup 1.0000x reference)
import functools

import jax
import jax.numpy as jnp
from jax import lax
from jax.experimental import pallas as pl
from jax.experimental.pallas import tpu as pltpu
from jax.experimental.pallas import tpu_sc as plsc

MAXD = 128
NBUK = 2 * MAXD + 1
H = 16
S = 2048
L = 16
NC, NS = 2, 16
HPC = H // NC
R = 16
RH = R // 2
PLEN = 4096
CHUNKS = PLEN // L
ROWS_PER_W = (H * S) // (NC * NS)
BLOCKS_PER_W = ROWS_PER_W // R
W_PIPE = 8

BAND_LO, BAND_HI = 119, 137
UNROLL = 7


def _body(weight_hbm, out_hbm, w_vmem, p_scr, p_sh, sem):
    cid = lax.axis_index("c")
    sid = lax.axis_index("s")
    hl = sid // 2
    h = cid * HPC + hl
    half = sid % 2

    pltpu.sync_copy(weight_hbm, w_vmem)

    lane = lax.iota(jnp.int32, L)
    h_vec = jnp.full((L,), h, dtype=jnp.int32)
    v_lo = plsc.load_gather(w_vmem, [jnp.zeros((L,), jnp.int32), h_vec])
    v_hi = plsc.load_gather(
        w_vmem, [jnp.full((L,), NBUK - 1, jnp.int32), h_vec]
    )

    d_base = half * RH
    for dl in range(RH):

        def flanks(ci, carry, dl=dl):
            for u in range(UNROLL):
                c_lo = ci * UNROLL + u
                p_scr[dl, pl.ds(c_lo * L, L)] = v_lo
                c_hi = BAND_HI + ci * UNROLL + u
                p_scr[dl, pl.ds(c_hi * L, L)] = v_hi
            return carry

        lax.fori_loop(0, BAND_LO // UNROLL, flanks, 0)

        def band(ci, carry, dl=dl):
            k = ci * L + lane + ((R - 1) - dl - (S - 1)) - d_base
            b = jnp.clip(k, -MAXD, MAXD) + MAXD
            p_scr[dl, pl.ds(ci * L, L)] = plsc.load_gather(w_vmem, [b, h_vec])
            return carry

        lax.fori_loop(BAND_LO, BAND_HI, band, 0)

    pltpu.sync_copy(p_scr, p_sh.at[hl, pl.ds(d_base, RH)])
    plsc.subcore_barrier()

    def fire(t, carry):
        m = half * BLOCKS_PER_W + t
        start = pl.multiple_of((2032 - 16 * m) + 0 * t, 16)
        src = p_sh.at[hl, :, pl.ds(start, S)]
        dst = out_hbm.at[h, pl.ds(R * m, R)]
        cp = pltpu.async_copy(src, dst, sem)

        @pl.when(t >= W_PIPE)
        def _():
            cp.wait()

        return carry

    lax.fori_loop(0, BLOCKS_PER_W, fire, 0)

    for _ in range(W_PIPE):
        pltpu.make_async_copy(
            out_hbm.at[0, pl.ds(0, R)], p_sh.at[0, :, pl.ds(0, S)], sem
        ).wait()


_bias = functools.partial(
    pl.kernel,
    out_type=jax.ShapeDtypeStruct((H, S, S), jnp.float32),
    mesh=plsc.VectorSubcoreMesh(
        core_axis_name="c", subcore_axis_name="s", num_cores=NC, num_subcores=NS
    ),
    scratch_types=[
        pltpu.VMEM((NBUK, H), jnp.float32),
        pltpu.VMEM((RH, PLEN), jnp.float32),
        pltpu.VMEM_SHARED((HPC, R, PLEN), jnp.float32),
        pltpu.SemaphoreType.DMA,
    ],
    compiler_params=pltpu.CompilerParams(
        use_tc_tiling_on_sc=False, needs_layout_passes=False
    ),
)(_body)


def kernel(seq_len, weight):
    del seq_len
    return _bias(weight)


# speedup vs baseline: 2.3688x; 2.3688x over previous
import functools

import jax
import jax.numpy as jnp
from jax import lax
from jax.experimental import pallas as pl
from jax.experimental.pallas import tpu as pltpu
from jax.experimental.pallas import tpu_sc as plsc

MAXD = 128
NBUK = 2 * MAXD + 1
H = 16
S = 2048
T = 128
NT = S // T
NSEL = 5
L = 16
NC, NS = 2, 16
R = 8
PLEN = 4096
W_PIPE = 4

BAND_LO, BAND_HI = 119, 137
UNROLL = 7
DMAS_PER_HEAD = NSEL * (T // R)
DMAS_PER_W = DMAS_PER_HEAD // 2


def _sc_body(weight_hbm, tiles_hbm, w_vmem, p_scr, sem):
    cid = lax.axis_index("c")
    sid = lax.axis_index("s")
    wid = sid * NC + cid
    h = wid // 2
    half = wid % 2

    pltpu.sync_copy(weight_hbm, w_vmem)

    lane = lax.iota(jnp.int32, L)
    h_vec = jnp.full((L,), h, dtype=jnp.int32)
    v_lo = plsc.load_gather(w_vmem, [jnp.zeros((L,), jnp.int32), h_vec])
    v_hi = plsc.load_gather(
        w_vmem, [jnp.full((L,), NBUK - 1, jnp.int32), h_vec]
    )

    for d in range(R):
        off = (R - 1) - d - (S - 1)

        def flanks(ci, carry, d=d):
            for u in range(UNROLL):
                c_lo = ci * UNROLL + u
                p_scr[d, pl.ds(c_lo * L, L)] = v_lo
                c_hi = BAND_HI + ci * UNROLL + u
                p_scr[d, pl.ds(c_hi * L, L)] = v_hi
            return carry

        lax.fori_loop(0, BAND_LO // UNROLL, flanks, 0)

        def band(ci, carry, d=d, off=off):
            k = ci * L + lane + off
            b = jnp.clip(k, -MAXD, MAXD) + MAXD
            p_scr[d, pl.ds(ci * L, L)] = plsc.load_gather(w_vmem, [b, h_vec])
            return carry

        lax.fori_loop(BAND_LO, BAND_HI, band, 0)

    def fire(t, carry):
        idx = half * DMAS_PER_W + t
        sel = idx // (T // R)
        m = idx % (T // R)
        start = pl.multiple_of(T * (sel - 2) + 2040 - R * m + 0 * t, 8)
        src = p_scr.at[:, pl.ds(start, T)]
        dst = tiles_hbm.at[h, sel, pl.ds(R * m, R)]
        cp = pltpu.async_copy(src, dst, sem)

        @pl.when(t >= W_PIPE)
        def _():
            cp.wait()

        return carry

    lax.fori_loop(0, DMAS_PER_W, fire, 0)

    for _ in range(W_PIPE):
        pltpu.make_async_copy(
            tiles_hbm.at[0, 0, pl.ds(0, R)], p_scr.at[:, pl.ds(0, T)], sem
        ).wait()


_sc_tiles = functools.partial(
    pl.kernel,
    out_type=jax.ShapeDtypeStruct((H, NSEL, T, T), jnp.float32),
    mesh=plsc.VectorSubcoreMesh(
        core_axis_name="c", subcore_axis_name="s", num_cores=NC, num_subcores=NS
    ),
    scratch_types=[
        pltpu.VMEM((NBUK, H), jnp.float32),
        pltpu.VMEM((R, PLEN), jnp.float32),
        pltpu.SemaphoreType.DMA,
    ],
    compiler_params=pltpu.CompilerParams(
        use_tc_tiling_on_sc=False, needs_layout_passes=False
    ),
)(_sc_body)


def _tc_body(tiles_ref, out_ref):
    ti = pl.program_id(1)
    for tj in range(NT):
        sel = jnp.clip(tj - ti, -2, 2) + 2
        out_ref[0, :, tj * T : (tj + 1) * T] = tiles_ref[0, sel]


def _tc_expand(tiles):
    return pl.pallas_call(
        _tc_body,
        grid=(H, NT),
        in_specs=[
            pl.BlockSpec((1, NSEL, T, T), lambda h, ti: (h, 0, 0, 0)),
        ],
        out_specs=pl.BlockSpec((1, T, S), lambda h, ti: (h, ti, 0)),
        out_shape=jax.ShapeDtypeStruct((H, S, S), jnp.float32),
    )(tiles)


def kernel(seq_len, weight):
    del seq_len
    return _tc_expand(_sc_tiles(weight))


# speedup vs baseline: 3.0081x; 1.2699x over previous
import functools

import jax
import jax.numpy as jnp
from jax import lax
from jax.experimental import pallas as pl
from jax.experimental.pallas import tpu as pltpu
from jax.experimental.pallas import tpu_sc as plsc

MAXD = 128
NBUK = 2 * MAXD + 1
H = 16
S = 2048
T = 128
NT = S // T
NSEL = 5
L = 16
NC, NS = 2, 16
R = 8
PLEN = 4096
W_PIPE = 4

BAND_LO, BAND_HI = 119, 137
UNROLL = 7
DMAS_PER_HEAD = NSEL * (T // R)
DMAS_PER_W = DMAS_PER_HEAD // 2


def _sc_body(weight_hbm, tiles_hbm, w_vmem, p_scr, sem):
    cid = lax.axis_index("c")
    sid = lax.axis_index("s")
    wid = sid * NC + cid
    h = wid // 2
    half = wid % 2

    pltpu.sync_copy(weight_hbm, w_vmem)

    lane = lax.iota(jnp.int32, L)
    h_vec = jnp.full((L,), h, dtype=jnp.int32)
    v_lo = plsc.load_gather(w_vmem, [jnp.zeros((L,), jnp.int32), h_vec])
    v_hi = plsc.load_gather(
        w_vmem, [jnp.full((L,), NBUK - 1, jnp.int32), h_vec]
    )

    for d in range(R):
        off = (R - 1) - d - (S - 1)

        def flanks(ci, carry, d=d):
            for u in range(UNROLL):
                c_lo = ci * UNROLL + u
                p_scr[d, pl.ds(c_lo * L, L)] = v_lo
                c_hi = BAND_HI + ci * UNROLL + u
                p_scr[d, pl.ds(c_hi * L, L)] = v_hi
            return carry

        lax.fori_loop(0, BAND_LO // UNROLL, flanks, 0)

        def band(ci, carry, d=d, off=off):
            k = ci * L + lane + off
            b = jnp.clip(k, -MAXD, MAXD) + MAXD
            p_scr[d, pl.ds(ci * L, L)] = plsc.load_gather(w_vmem, [b, h_vec])
            return carry

        lax.fori_loop(BAND_LO, BAND_HI, band, 0)

    def fire(t, carry):
        idx = half * DMAS_PER_W + t
        sel = idx // (T // R)
        m = idx % (T // R)
        start = pl.multiple_of(T * (sel - 2) + 2040 - R * m + 0 * t, 8)
        src = p_scr.at[:, pl.ds(start, T)]
        dst = tiles_hbm.at[h, sel, pl.ds(R * m, R)]
        cp = pltpu.async_copy(src, dst, sem)

        @pl.when(t >= W_PIPE)
        def _():
            cp.wait()

        return carry

    lax.fori_loop(0, DMAS_PER_W, fire, 0)

    for _ in range(W_PIPE):
        pltpu.make_async_copy(
            tiles_hbm.at[0, 0, pl.ds(0, R)], p_scr.at[:, pl.ds(0, T)], sem
        ).wait()


_sc_tiles = functools.partial(
    pl.kernel,
    out_type=jax.ShapeDtypeStruct((H, NSEL, T, T), jnp.float32),
    mesh=plsc.VectorSubcoreMesh(
        core_axis_name="c", subcore_axis_name="s", num_cores=NC, num_subcores=NS
    ),
    scratch_types=[
        pltpu.VMEM((NBUK, H), jnp.float32),
        pltpu.VMEM((R, PLEN), jnp.float32),
        pltpu.SemaphoreType.DMA,
    ],
    compiler_params=pltpu.CompilerParams(
        use_tc_tiling_on_sc=False, needs_layout_passes=False
    ),
)(_sc_body)


TI_PER_BLK = 2


def _tc_body(tiles_ref, out_ref):
    tb = pl.program_id(1)
    for tl in range(TI_PER_BLK):
        ti = tb * TI_PER_BLK + tl
        for tj in range(NT):
            sel = jnp.clip(tj - ti, -2, 2) + 2
            out_ref[0, tl * T : (tl + 1) * T, tj * T : (tj + 1) * T] = (
                tiles_ref[0, sel]
            )


def _tc_expand(tiles):
    return pl.pallas_call(
        _tc_body,
        grid=(H, NT // TI_PER_BLK),
        in_specs=[
            pl.BlockSpec((1, NSEL, T, T), lambda h, tb: (h, 0, 0, 0)),
        ],
        out_specs=pl.BlockSpec((1, TI_PER_BLK * T, S), lambda h, tb: (h, tb, 0)),
        out_shape=jax.ShapeDtypeStruct((H, S, S), jnp.float32),
    )(tiles)


def kernel(seq_len, weight):
    del seq_len
    return _tc_expand(_sc_tiles(weight))


# speedup vs baseline: 3.5595x; 1.1833x over previous
import functools

import jax
import jax.numpy as jnp
from jax import lax
from jax.experimental import pallas as pl
from jax.experimental.pallas import tpu as pltpu
from jax.experimental.pallas import tpu_sc as plsc

MAXD = 128
NBUK = 2 * MAXD + 1
H = 16
S = 2048
T = 128
NT = S // T
NSEL = 5
L = 16
NC, NS = 2, 16
R = 8
PLEN = 4096
W_PIPE = 4

BAND_LO, BAND_HI = 119, 137
UNROLL = 7
DMAS_PER_HEAD = NSEL * (T // R)
DMAS_PER_W = DMAS_PER_HEAD // 2


def _sc_body(weight_hbm, tiles_hbm, w_vmem, p_scr, sem):
    cid = lax.axis_index("c")
    sid = lax.axis_index("s")
    wid = sid * NC + cid
    h = wid // 2
    half = wid % 2

    pltpu.sync_copy(weight_hbm, w_vmem)

    lane = lax.iota(jnp.int32, L)
    h_vec = jnp.full((L,), h, dtype=jnp.int32)
    v_lo = plsc.load_gather(w_vmem, [jnp.zeros((L,), jnp.int32), h_vec])
    v_hi = plsc.load_gather(
        w_vmem, [jnp.full((L,), NBUK - 1, jnp.int32), h_vec]
    )

    for d in range(R):
        off = (R - 1) - d - (S - 1)

        def flanks(ci, carry, d=d):
            for u in range(UNROLL):
                c_lo = ci * UNROLL + u
                p_scr[d, pl.ds(c_lo * L, L)] = v_lo
                c_hi = BAND_HI + ci * UNROLL + u
                p_scr[d, pl.ds(c_hi * L, L)] = v_hi
            return carry

        lax.fori_loop(0, BAND_LO // UNROLL, flanks, 0)

        def band(ci, carry, d=d, off=off):
            k = ci * L + lane + off
            b = jnp.clip(k, -MAXD, MAXD) + MAXD
            p_scr[d, pl.ds(ci * L, L)] = plsc.load_gather(w_vmem, [b, h_vec])
            return carry

        lax.fori_loop(BAND_LO, BAND_HI, band, 0)

    def fire(t, carry):
        idx = half * DMAS_PER_W + t
        sel = idx // (T // R)
        m = idx % (T // R)
        start = pl.multiple_of(T * (sel - 2) + 2040 - R * m + 0 * t, 8)
        src = p_scr.at[:, pl.ds(start, T)]
        dst = tiles_hbm.at[h, sel, pl.ds(R * m, R)]
        cp = pltpu.async_copy(src, dst, sem)

        @pl.when(t >= W_PIPE)
        def _():
            cp.wait()

        return carry

    lax.fori_loop(0, DMAS_PER_W, fire, 0)

    for _ in range(W_PIPE):
        pltpu.make_async_copy(
            tiles_hbm.at[0, 0, pl.ds(0, R)], p_scr.at[:, pl.ds(0, T)], sem
        ).wait()


_sc_tiles = functools.partial(
    pl.kernel,
    out_type=jax.ShapeDtypeStruct((H, NSEL, T, T), jnp.float32),
    mesh=plsc.VectorSubcoreMesh(
        core_axis_name="c", subcore_axis_name="s", num_cores=NC, num_subcores=NS
    ),
    scratch_types=[
        pltpu.VMEM((NBUK, H), jnp.float32),
        pltpu.VMEM((R, PLEN), jnp.float32),
        pltpu.SemaphoreType.DMA,
    ],
    compiler_params=pltpu.CompilerParams(
        use_tc_tiling_on_sc=False, needs_layout_passes=False
    ),
)(_sc_body)


TI_PER_BLK = 4


def _tc_body(tiles_ref, out_ref):
    tb = pl.program_id(1)
    for tl in range(TI_PER_BLK):
        ti = tb * TI_PER_BLK + tl
        for tj in range(NT):
            sel = jnp.clip(tj - ti, -2, 2) + 2
            out_ref[0, tl * T : (tl + 1) * T, tj * T : (tj + 1) * T] = (
                tiles_ref[0, sel]
            )


def _tc_expand(tiles):
    return pl.pallas_call(
        _tc_body,
        grid=(H, NT // TI_PER_BLK),
        in_specs=[
            pl.BlockSpec((1, NSEL, T, T), lambda h, tb: (h, 0, 0, 0)),
        ],
        out_specs=pl.BlockSpec((1, TI_PER_BLK * T, S), lambda h, tb: (h, tb, 0)),
        out_shape=jax.ShapeDtypeStruct((H, S, S), jnp.float32),
    )(tiles)


def kernel(seq_len, weight):
    del seq_len
    return _tc_expand(_sc_tiles(weight))


# speedup vs baseline: 3.9180x; 1.1007x over previous
import functools

import jax
import jax.numpy as jnp
from jax import lax
from jax.experimental import pallas as pl
from jax.experimental.pallas import tpu as pltpu
from jax.experimental.pallas import tpu_sc as plsc

MAXD = 128
NBUK = 2 * MAXD + 1
H = 16
S = 2048
T = 128
NT = S // T
NSEL = 5
L = 16
NC, NS = 2, 16
R = 8
PLEN = 4096
W_PIPE = 4

BAND_LO, BAND_HI = 119, 137
UNROLL = 7
DMAS_PER_HEAD = NSEL * (T // R)
DMAS_PER_W = DMAS_PER_HEAD // 2


def _sc_body(weight_hbm, tiles_hbm, w_vmem, p_scr, sem):
    cid = lax.axis_index("c")
    sid = lax.axis_index("s")
    wid = sid * NC + cid
    h = wid // 2
    half = wid % 2

    pltpu.sync_copy(weight_hbm, w_vmem)

    lane = lax.iota(jnp.int32, L)
    h_vec = jnp.full((L,), h, dtype=jnp.int32)
    v_lo = plsc.load_gather(w_vmem, [jnp.zeros((L,), jnp.int32), h_vec])
    v_hi = plsc.load_gather(
        w_vmem, [jnp.full((L,), NBUK - 1, jnp.int32), h_vec]
    )

    for d in range(R):
        off = (R - 1) - d - (S - 1)

        def flanks(ci, carry, d=d):
            for u in range(UNROLL):
                c_lo = ci * UNROLL + u
                p_scr[d, pl.ds(c_lo * L, L)] = v_lo
                c_hi = BAND_HI + ci * UNROLL + u
                p_scr[d, pl.ds(c_hi * L, L)] = v_hi
            return carry

        lax.fori_loop(0, BAND_LO // UNROLL, flanks, 0)

        def band(ci, carry, d=d, off=off):
            k = ci * L + lane + off
            b = jnp.clip(k, -MAXD, MAXD) + MAXD
            p_scr[d, pl.ds(ci * L, L)] = plsc.load_gather(w_vmem, [b, h_vec])
            return carry

        lax.fori_loop(BAND_LO, BAND_HI, band, 0)

    def fire(t, carry):
        idx = half * DMAS_PER_W + t
        sel = idx // (T // R)
        m = idx % (T // R)
        start = pl.multiple_of(T * (sel - 2) + 2040 - R * m + 0 * t, 8)
        src = p_scr.at[:, pl.ds(start, T)]
        dst = tiles_hbm.at[h, sel, pl.ds(R * m, R)]
        cp = pltpu.async_copy(src, dst, sem)

        @pl.when(t >= W_PIPE)
        def _():
            cp.wait()

        return carry

    lax.fori_loop(0, DMAS_PER_W, fire, 0)

    for _ in range(W_PIPE):
        pltpu.make_async_copy(
            tiles_hbm.at[0, 0, pl.ds(0, R)], p_scr.at[:, pl.ds(0, T)], sem
        ).wait()


_sc_tiles = functools.partial(
    pl.kernel,
    out_type=jax.ShapeDtypeStruct((H, NSEL, T, T), jnp.float32),
    mesh=plsc.VectorSubcoreMesh(
        core_axis_name="c", subcore_axis_name="s", num_cores=NC, num_subcores=NS
    ),
    scratch_types=[
        pltpu.VMEM((NBUK, H), jnp.float32),
        pltpu.VMEM((R, PLEN), jnp.float32),
        pltpu.SemaphoreType.DMA,
    ],
    compiler_params=pltpu.CompilerParams(
        use_tc_tiling_on_sc=False, needs_layout_passes=False
    ),
)(_sc_body)


TI_PER_BLK = 8


def _tc_body(tiles_ref, out_ref):
    tb = pl.program_id(1)
    for tl in range(TI_PER_BLK):
        ti = tb * TI_PER_BLK + tl
        for tj in range(NT):
            sel = jnp.clip(tj - ti, -2, 2) + 2
            out_ref[0, tl * T : (tl + 1) * T, tj * T : (tj + 1) * T] = (
                tiles_ref[0, sel]
            )


def _tc_expand(tiles):
    return pl.pallas_call(
        _tc_body,
        grid=(H, NT // TI_PER_BLK),
        in_specs=[
            pl.BlockSpec((1, NSEL, T, T), lambda h, tb: (h, 0, 0, 0)),
        ],
        out_specs=pl.BlockSpec((1, TI_PER_BLK * T, S), lambda h, tb: (h, tb, 0)),
        out_shape=jax.ShapeDtypeStruct((H, S, S), jnp.float32),
    )(tiles)


def kernel(seq_len, weight):
    del seq_len
    return _tc_expand(_sc_tiles(weight))
